# Initial kernel scaffold; baseline (speedup 1.0000x reference)
import jax
import jax.numpy as jnp
from jax.experimental import pallas as pl


def kernel(x, edge_index, W1, b1, Wq, bq, Wk, bk, Wv, bv, W2, b2):
    raise NotImplementedError("write your pallas kernel here")


# speedup vs baseline: 2.4072x; 2.4072x over previous
import math

import jax
import jax.numpy as jnp
from jax import lax
from jax.experimental import pallas as pl

N = 10000
E = 160000
C = 128
NPAD = 10240
BR = 256
EBLK = 256
INV_SQRT_DH = 1.0 / math.sqrt(128.0)
F32 = jnp.float32


def _tck1_body(x_ref, w1_ref, b1_ref, wv0_ref, bv0_ref, h_ref, va_ref):
    h = jnp.maximum(
        jnp.dot(x_ref[...], w1_ref[...], preferred_element_type=F32)
        + b1_ref[...], 0.0)
    h_ref[...] = h
    va_ref[...] = (jnp.dot(h, wv0_ref[...], preferred_element_type=F32)
                   + bv0_ref[...])


_tck1 = pl.pallas_call(
    _tck1_body,
    grid=(NPAD // BR,),
    in_specs=[
        pl.BlockSpec((BR, C), lambda i: (i, 0)),
        pl.BlockSpec((C, C), lambda i: (0, 0)),
        pl.BlockSpec((1, C), lambda i: (0, 0)),
        pl.BlockSpec((C, C), lambda i: (0, 0)),
        pl.BlockSpec((1, C), lambda i: (0, 0)),
    ],
    out_specs=[pl.BlockSpec((BR, C), lambda i: (i, 0))] * 2,
    out_shape=[jax.ShapeDtypeStruct((NPAD, C), F32)] * 2,
)


def _tck2_body(deg_ref, va_ref, dis_ref, vta_ref):
    dis = lax.rsqrt(deg_ref[...])
    dis_ref[...] = dis
    vta_ref[...] = dis[:, None] * va_ref[...]


_tck2 = pl.pallas_call(
    _tck2_body,
    grid=(NPAD // BR,),
    in_specs=[
        pl.BlockSpec((BR,), lambda i: (i,)),
        pl.BlockSpec((BR, C), lambda i: (i, 0)),
    ],
    out_specs=[
        pl.BlockSpec((BR,), lambda i: (i,)),
        pl.BlockSpec((BR, C), lambda i: (i, 0)),
    ],
    out_shape=[
        jax.ShapeDtypeStruct((NPAD,), F32),
        jax.ShapeDtypeStruct((NPAD, C), F32),
    ],
)


def _tck3_body(h_ref, va_ref, dis_ref, accv_ref,
               wq_ref, bq_ref, wk_ref, wv_ref, bv_ref,
               qt_ref, kd_ref, v0_ref, vd_ref):
    dis = dis_ref[...]
    h = h_ref[...]
    out1 = dis[:, None] * (accv_ref[...] + dis[:, None] * va_ref[...])
    h1 = jnp.maximum(out1, 0.0)
    qt_ref[...] = (jnp.dot(h1, wq_ref[...], preferred_element_type=F32)
                   + bq_ref[...]) * INV_SQRT_DH
    g = h1 - h
    kd_ref[...] = jnp.dot(g, wk_ref[...], preferred_element_type=F32)
    v0_ref[...] = dis[:, None] * (
        jnp.dot(h, wv_ref[...], preferred_element_type=F32) + bv_ref[...])
    vd_ref[...] = dis[:, None] * jnp.dot(
        g, wv_ref[...], preferred_element_type=F32)


_tck3 = pl.pallas_call(
    _tck3_body,
    grid=(NPAD // BR,),
    in_specs=[
        pl.BlockSpec((BR, C), lambda i: (i, 0)),
        pl.BlockSpec((BR, C), lambda i: (i, 0)),
        pl.BlockSpec((BR,), lambda i: (i,)),
        pl.BlockSpec((BR, C), lambda i: (i, 0)),
        pl.BlockSpec((C, C), lambda i: (0, 0)),
        pl.BlockSpec((1, C), lambda i: (0, 0)),
        pl.BlockSpec((C, C), lambda i: (0, 0)),
        pl.BlockSpec((C, C), lambda i: (0, 0)),
        pl.BlockSpec((1, C), lambda i: (0, 0)),
    ],
    out_specs=[pl.BlockSpec((BR, C), lambda i: (i, 0))] * 4,
    out_shape=[jax.ShapeDtypeStruct((NPAD, C), F32)] * 4,
)


def _edge_body(qg_ref, kg_ref, v0g_ref, vdg_ref, msg_ref):
    s = jnp.sum(qg_ref[...] * kg_ref[...], axis=1)
    a1 = 1.0 / (1.0 + jnp.exp(-s))
    msg_ref[...] = v0g_ref[...] + a1[:, None] * vdg_ref[...]


_tck_edge = pl.pallas_call(
    _edge_body,
    grid=(E // EBLK,),
    in_specs=[pl.BlockSpec((EBLK, C), lambda i: (i, 0))] * 4,
    out_specs=pl.BlockSpec((EBLK, C), lambda i: (i, 0)),
    out_shape=jax.ShapeDtypeStruct((E, C), F32),
)


def _tck4_body(dis_ref, qt_ref, kd_ref, v0_ref, vd_ref, acc2_ref,
               w2_ref, b2_ref, z_ref):
    dis = dis_ref[...]
    s = jnp.sum(qt_ref[...] * kd_ref[...], axis=1)
    a1 = 1.0 / (1.0 + jnp.exp(-s))
    out2 = dis[:, None] * (acc2_ref[...] + v0_ref[...]
                           + a1[:, None] * vd_ref[...])
    h2 = jnp.maximum(out2, 0.0)
    z = jnp.dot(h2, w2_ref[...], preferred_element_type=F32) + b2_ref[...]
    z = z - jnp.max(z, axis=1, keepdims=True)
    z_ref[...] = z - jnp.log(jnp.sum(jnp.exp(z), axis=1, keepdims=True))


_tck4 = pl.pallas_call(
    _tck4_body,
    grid=(NPAD // BR,),
    in_specs=[
        pl.BlockSpec((BR,), lambda i: (i,)),
        pl.BlockSpec((BR, C), lambda i: (i, 0)),
        pl.BlockSpec((BR, C), lambda i: (i, 0)),
        pl.BlockSpec((BR, C), lambda i: (i, 0)),
        pl.BlockSpec((BR, C), lambda i: (i, 0)),
        pl.BlockSpec((BR, C), lambda i: (i, 0)),
        pl.BlockSpec((C, 64), lambda i: (0, 0)),
        pl.BlockSpec((1, 64), lambda i: (0, 0)),
    ],
    out_specs=pl.BlockSpec((BR, 64), lambda i: (i, 0)),
    out_shape=jax.ShapeDtypeStruct((NPAD, 64), F32),
)


def _padn(a):
    return jnp.pad(a, ((0, NPAD - N),) + ((0, 0),) * (a.ndim - 1))


def kernel(x, edge_index, W1, b1, Wq, bq, Wk, bk, Wv, bv, W2, b2):
    row = edge_index[0].astype(jnp.int32)
    col = edge_index[1].astype(jnp.int32)

    H, Va = _tck1(_padn(x), W1, b1.reshape(1, C), Wv[0], bv[0].reshape(1, C))

    deg = jnp.ones((N,), F32).at[col].add(1.0)
    deg = jnp.pad(deg, (0, NPAD - N), constant_values=1.0)
    dis, Vta = _tck2(deg, Va)

    accV = jnp.zeros((N, C), F32).at[col].add(Vta[row])
    Qt, KD, V0t, VDt = _tck3(H, Va, dis, _padn(accV),
                             Wq[1], bq[1].reshape(1, C), Wk[1], Wv[1],
                             bv[1].reshape(1, C))

    msg = _tck_edge(Qt[col], KD[row], V0t[row], VDt[row])
    acc2 = jnp.zeros((N, C), F32).at[col].add(msg)

    z = _tck4(dis, Qt, KD, V0t, VDt, _padn(acc2), W2, b2.reshape(1, 64))
    return z[:N]


# speedup vs baseline: 2.8101x; 1.1674x over previous
import math

import jax
import jax.numpy as jnp
from jax import lax
from jax.experimental import pallas as pl

N = 10000
E = 160000
C = 128
NPAD = 10240
BR = 256
EBLK = 256
INV_SQRT_DH = 1.0 / math.sqrt(128.0)
F32 = jnp.float32


def _tck1_body(x_ref, w1_ref, b1_ref, wv0_ref, bv0_ref, h_ref, va_ref):
    h = jnp.maximum(
        jnp.dot(x_ref[...], w1_ref[...], preferred_element_type=F32)
        + b1_ref[...], 0.0)
    h_ref[...] = h
    va_ref[...] = (jnp.dot(h, wv0_ref[...], preferred_element_type=F32)
                   + bv0_ref[...])


_tck1 = pl.pallas_call(
    _tck1_body,
    grid=(NPAD // BR,),
    in_specs=[
        pl.BlockSpec((BR, C), lambda i: (i, 0)),
        pl.BlockSpec((C, C), lambda i: (0, 0)),
        pl.BlockSpec((1, C), lambda i: (0, 0)),
        pl.BlockSpec((C, C), lambda i: (0, 0)),
        pl.BlockSpec((1, C), lambda i: (0, 0)),
    ],
    out_specs=[pl.BlockSpec((BR, C), lambda i: (i, 0))] * 2,
    out_shape=[jax.ShapeDtypeStruct((NPAD, C), F32)] * 2,
)


def _tck2_body(deg_ref, va_ref, dis_ref, vta_ref):
    dis = lax.rsqrt(deg_ref[...])
    dis_ref[...] = dis
    vta_ref[...] = dis[:, None] * va_ref[...]


_tck2 = pl.pallas_call(
    _tck2_body,
    grid=(NPAD // BR,),
    in_specs=[
        pl.BlockSpec((BR,), lambda i: (i,)),
        pl.BlockSpec((BR, C), lambda i: (i, 0)),
    ],
    out_specs=[
        pl.BlockSpec((BR,), lambda i: (i,)),
        pl.BlockSpec((BR, C), lambda i: (i, 0)),
    ],
    out_shape=[
        jax.ShapeDtypeStruct((NPAD,), F32),
        jax.ShapeDtypeStruct((NPAD, C), F32),
    ],
)


def _tck3_body(h_ref, va_ref, dis_ref, accv_ref,
               wq_ref, bq_ref, wk_ref, wv_ref, bv_ref,
               qt_ref, src_ref):
    dis = dis_ref[...]
    h = h_ref[...]
    out1 = dis[:, None] * (accv_ref[...] + dis[:, None] * va_ref[...])
    h1 = jnp.maximum(out1, 0.0)
    qt_ref[...] = (jnp.dot(h1, wq_ref[...], preferred_element_type=F32)
                   + bq_ref[...]) * INV_SQRT_DH
    g = h1 - h
    src_ref[:, 0:C] = jnp.dot(g, wk_ref[...], preferred_element_type=F32)
    src_ref[:, C:2 * C] = dis[:, None] * (
        jnp.dot(h, wv_ref[...], preferred_element_type=F32) + bv_ref[...])
    src_ref[:, 2 * C:3 * C] = dis[:, None] * jnp.dot(
        g, wv_ref[...], preferred_element_type=F32)


_tck3 = pl.pallas_call(
    _tck3_body,
    grid=(NPAD // BR,),
    in_specs=[
        pl.BlockSpec((BR, C), lambda i: (i, 0)),
        pl.BlockSpec((BR, C), lambda i: (i, 0)),
        pl.BlockSpec((BR,), lambda i: (i,)),
        pl.BlockSpec((BR, C), lambda i: (i, 0)),
        pl.BlockSpec((C, C), lambda i: (0, 0)),
        pl.BlockSpec((1, C), lambda i: (0, 0)),
        pl.BlockSpec((C, C), lambda i: (0, 0)),
        pl.BlockSpec((C, C), lambda i: (0, 0)),
        pl.BlockSpec((1, C), lambda i: (0, 0)),
    ],
    out_specs=[
        pl.BlockSpec((BR, C), lambda i: (i, 0)),
        pl.BlockSpec((BR, 3 * C), lambda i: (i, 0)),
    ],
    out_shape=[
        jax.ShapeDtypeStruct((NPAD, C), F32),
        jax.ShapeDtypeStruct((NPAD, 3 * C), F32),
    ],
)


def _edge_body(qg_ref, sg_ref, msg_ref):
    s = jnp.sum(qg_ref[...] * sg_ref[:, 0:C], axis=1)
    a1 = 1.0 / (1.0 + jnp.exp(-s))
    msg_ref[...] = (sg_ref[:, C:2 * C]
                    + a1[:, None] * sg_ref[:, 2 * C:3 * C])


_tck_edge = pl.pallas_call(
    _edge_body,
    grid=(E // EBLK,),
    in_specs=[
        pl.BlockSpec((EBLK, C), lambda i: (i, 0)),
        pl.BlockSpec((EBLK, 3 * C), lambda i: (i, 0)),
    ],
    out_specs=pl.BlockSpec((EBLK, C), lambda i: (i, 0)),
    out_shape=jax.ShapeDtypeStruct((E, C), F32),
)


def _tck4_body(dis_ref, qt_ref, src_ref, acc2_ref,
               w2_ref, b2_ref, z_ref):
    dis = dis_ref[...]
    s = jnp.sum(qt_ref[...] * src_ref[:, 0:C], axis=1)
    a1 = 1.0 / (1.0 + jnp.exp(-s))
    out2 = dis[:, None] * (acc2_ref[...] + src_ref[:, C:2 * C]
                           + a1[:, None] * src_ref[:, 2 * C:3 * C])
    h2 = jnp.maximum(out2, 0.0)
    z = jnp.dot(h2, w2_ref[...], preferred_element_type=F32) + b2_ref[...]
    z = z - jnp.max(z, axis=1, keepdims=True)
    z_ref[...] = z - jnp.log(jnp.sum(jnp.exp(z), axis=1, keepdims=True))


_tck4 = pl.pallas_call(
    _tck4_body,
    grid=(NPAD // BR,),
    in_specs=[
        pl.BlockSpec((BR,), lambda i: (i,)),
        pl.BlockSpec((BR, C), lambda i: (i, 0)),
        pl.BlockSpec((BR, 3 * C), lambda i: (i, 0)),
        pl.BlockSpec((BR, C), lambda i: (i, 0)),
        pl.BlockSpec((C, 64), lambda i: (0, 0)),
        pl.BlockSpec((1, 64), lambda i: (0, 0)),
    ],
    out_specs=pl.BlockSpec((BR, 64), lambda i: (i, 0)),
    out_shape=jax.ShapeDtypeStruct((NPAD, 64), F32),
)


def _padn(a):
    return jnp.pad(a, ((0, NPAD - N),) + ((0, 0),) * (a.ndim - 1))


def kernel(x, edge_index, W1, b1, Wq, bq, Wk, bk, Wv, bv, W2, b2):
    row = edge_index[0].astype(jnp.int32)
    col = edge_index[1].astype(jnp.int32)

    H, Va = _tck1(_padn(x), W1, b1.reshape(1, C), Wv[0], bv[0].reshape(1, C))

    deg = jnp.ones((N,), F32).at[col].add(1.0)
    deg = jnp.pad(deg, (0, NPAD - N), constant_values=1.0)
    dis, Vta = _tck2(deg, Va)

    accV = jnp.zeros((N, C), F32).at[col].add(Vta[row])
    Qt, Src = _tck3(H, Va, dis, _padn(accV),
                    Wq[1], bq[1].reshape(1, C), Wk[1], Wv[1],
                    bv[1].reshape(1, C))

    msg = _tck_edge(Qt[col], Src[row])
    acc2 = jnp.zeros((N, C), F32).at[col].add(msg)

    z = _tck4(dis, Qt, Src, _padn(acc2), W2, b2.reshape(1, 64))
    return z[:N]
